# baseline (speedup 1.0000x reference)
import functools

import jax
import jax.numpy as jnp
from jax import lax
from jax.experimental import pallas as pl
from jax.experimental.pallas import tpu as pltpu

_EPS = 1e-12


def _round_up(v, n):
    return (v + n - 1) // n * n


def _cosface_fused_kernel(lab_ref, x_ref, w_ref, o_ref, *, s, m, tile_c):
    x = x_ref[...]
    sx = jnp.sum(x * x, axis=1, keepdims=True)
    inv_nx = lax.rsqrt(jnp.maximum(sx, _EPS * _EPS)) * s
    xn = (x * inv_nx).astype(jnp.bfloat16)

    w = w_ref[...]
    sw = jnp.sum(w * w, axis=1, keepdims=True)
    inv_nw = lax.rsqrt(jnp.maximum(sw, _EPS * _EPS))
    wn = (w * inv_nw).astype(jnp.bfloat16)

    raw = lax.dot_general(
        xn, wn,
        dimension_numbers=(((1,), (1,)), ((), ())),
        preferred_element_type=jnp.float32)

    col0 = pl.program_id(0) * tile_c
    class_ids = lax.broadcasted_iota(jnp.int32, raw.shape, 1) + col0
    labels = lab_ref[...]
    o_ref[...] = jnp.where(class_ids == labels, raw - (s * m), raw)


def kernel(x, W, label, s=30.0, m=0.35, tile_c=1280):
    B, F = x.shape
    C, F2 = W.shape
    assert F == F2

    tc = tile_c if C >= tile_c else _round_up(C, 128)
    Bp = _round_up(B, 8)
    Cp = _round_up(C, tc)
    x_p = x if Bp == B else jnp.pad(x, ((0, Bp - B), (0, 0)))
    W_p = W if Cp == C else jnp.pad(W, ((0, Cp - C), (0, 0)))
    lab = label.astype(jnp.int32).reshape(B, 1)
    lab_p = lab if Bp == B else jnp.pad(lab, ((0, Bp - B), (0, 0)),
                                        constant_values=-1)

    out = pl.pallas_call(
        functools.partial(_cosface_fused_kernel, s=s, m=m, tile_c=tc),
        out_shape=jax.ShapeDtypeStruct((Bp, Cp), jnp.float32),
        grid=(Cp // tc,),
        in_specs=[
            pl.BlockSpec((Bp, 1), lambda j: (0, 0)),
            pl.BlockSpec((Bp, F), lambda j: (0, 0)),
            pl.BlockSpec((tc, F), lambda j: (j, 0)),
        ],
        out_specs=pl.BlockSpec((Bp, tc), lambda j: (0, j)),
        compiler_params=pltpu.CompilerParams(
            dimension_semantics=("parallel",),
            vmem_limit_bytes=48 * 1024 * 1024,
        ),
    )(lab_p, x_p, W_p)
    return out[:B, :C]


# speedup vs baseline: 1.1433x; 1.1433x over previous
import functools

import jax
import jax.numpy as jnp
from jax import lax
from jax.experimental import pallas as pl
from jax.experimental.pallas import tpu as pltpu

_EPS = 1e-12


def _round_up(v, n):
    return (v + n - 1) // n * n


def _cosface_fused_kernel(lab_ref, x_ref, w_ref, o_ref, *, s, m, tile_c):
    x = x_ref[...]
    sx = jnp.sum(x * x, axis=1, keepdims=True)
    inv_nx = lax.rsqrt(jnp.maximum(sx, _EPS * _EPS)) * s
    xn = (x * inv_nx).astype(jnp.bfloat16)

    w = w_ref[...]
    sw = jnp.sum(w * w, axis=1, keepdims=True)
    inv_nw = lax.rsqrt(jnp.maximum(sw, _EPS * _EPS))
    wn = (w * inv_nw).astype(jnp.bfloat16)

    raw = lax.dot_general(
        xn, wn,
        dimension_numbers=(((1,), (1,)), ((), ())),
        preferred_element_type=jnp.float32)

    col0 = pl.program_id(0) * tile_c
    class_ids = lax.broadcasted_iota(jnp.int32, raw.shape, 1) + col0
    labels = lab_ref[...]
    o_ref[...] = jnp.where(class_ids == labels, raw - (s * m), raw)


def kernel(x, W, label, s=30.0, m=0.35, tile_c=5120):
    B, F = x.shape
    C, F2 = W.shape
    assert F == F2

    tc = tile_c if C >= tile_c else _round_up(C, 128)
    Bp = _round_up(B, 8)
    Cp = _round_up(C, tc)
    x_p = x if Bp == B else jnp.pad(x, ((0, Bp - B), (0, 0)))
    W_p = W if Cp == C else jnp.pad(W, ((0, Cp - C), (0, 0)))
    lab = label.astype(jnp.int32).reshape(B, 1)
    lab_p = lab if Bp == B else jnp.pad(lab, ((0, Bp - B), (0, 0)),
                                        constant_values=-1)

    out = pl.pallas_call(
        functools.partial(_cosface_fused_kernel, s=s, m=m, tile_c=tc),
        out_shape=jax.ShapeDtypeStruct((Bp, Cp), jnp.float32),
        grid=(Cp // tc,),
        in_specs=[
            pl.BlockSpec((Bp, 1), lambda j: (0, 0)),
            pl.BlockSpec((Bp, F), lambda j: (0, 0)),
            pl.BlockSpec((tc, F), lambda j: (j, 0)),
        ],
        out_specs=pl.BlockSpec((Bp, tc), lambda j: (0, j)),
        compiler_params=pltpu.CompilerParams(
            dimension_semantics=("parallel",),
            vmem_limit_bytes=58 * 1024 * 1024,
        ),
    )(lab_p, x_p, W_p)
    return out[:B, :C]
